# baseline (speedup 1.0000x reference)
import functools

import jax
import jax.numpy as jnp
from jax import lax
from jax.experimental import pallas as pl
from jax.experimental.pallas import tpu as pltpu
from jax.experimental.pallas import tpu_sc as plsc

N = 10000
E = 320000
H = 128
C = 40
BN_EPS = 1e-5

NC = 2
NS = 16
NW = NC * NS
EPW = E // NW
CHUNK = 128
FULL_CHUNKS = EPW // CHUNK
REM = EPW - FULL_CHUNKS * CHUNK
N_PAD = 10112
ROWS_PER_TILE = N_PAD // NS


def _agg_body(x_hbm, src_hbm, dst_hbm, zeros_hbm, out_hbm,
              agg_sh, src_v0, dst_v0, src_v1, dst_v1,
              rows_v0, rows_v1, srcr_v, dstr_v, rowsr_v,
              sem_g0, sem_g1, sem_i0, sem_i1, sem_r):
    cid = lax.axis_index("c")
    sid = lax.axis_index("s")
    wid = sid * NC + cid
    base = wid * EPW

    src_b = (src_v0, src_v1)
    dst_b = (dst_v0, dst_v1)
    rows_b = (rows_v0, rows_v1)
    sem_g = (sem_g0, sem_g1)
    sem_i = (sem_i0, sem_i1)

    def issue_idx(c, b):
        off = base + c * CHUNK
        pltpu.async_copy(src_hbm.at[pl.ds(off, CHUNK)], src_b[b], sem_i[b])
        pltpu.async_copy(dst_hbm.at[pl.ds(off, CHUNK)], dst_b[b], sem_i[b])

    def wait_idx(c, b):
        off = base + c * CHUNK
        pltpu.make_async_copy(src_hbm.at[pl.ds(off, CHUNK)], src_b[b], sem_i[b]).wait()
        pltpu.make_async_copy(dst_hbm.at[pl.ds(off, CHUNK)], dst_b[b], sem_i[b]).wait()

    def start_gather(b):
        pltpu.async_copy(x_hbm.at[src_b[b]], rows_b[b], sem_g[b])

    def wait_gather(b):
        pltpu.make_async_copy(x_hbm.at[src_b[b]], rows_b[b], sem_g[b]).wait()

    def scatter(b):
        pltpu.sync_copy(rows_b[b], agg_sh.at[dst_b[b]], add=True)

    issue_idx(0, 0)
    issue_idx(1, 1)
    pltpu.sync_copy(zeros_hbm.at[pl.ds(sid * ROWS_PER_TILE, ROWS_PER_TILE)],
                    agg_sh.at[pl.ds(sid * ROWS_PER_TILE, ROWS_PER_TILE)])
    plsc.subcore_barrier()
    wait_idx(0, 0)
    start_gather(0)

    def half_step(c, b):
        wait_idx(c + 1, 1 - b)
        start_gather(1 - b)
        wait_gather(b)
        scatter(b)
        issue_idx(c + 2, b)

    def pair_step(i, carry):
        half_step(2 * i, 0)
        half_step(2 * i + 1, 1)
        return carry

    lax.fori_loop(0, FULL_CHUNKS // 2 - 1, pair_step, 0)

    wait_idx(FULL_CHUNKS - 1, 1)
    start_gather(1)
    wait_gather(0)
    scatter(0)
    wait_gather(1)
    scatter(1)

    off = base + FULL_CHUNKS * CHUNK
    pltpu.sync_copy(src_hbm.at[pl.ds(off, REM)], srcr_v)
    pltpu.sync_copy(dst_hbm.at[pl.ds(off, REM)], dstr_v)
    pltpu.async_copy(x_hbm.at[srcr_v], rowsr_v, sem_r).wait()
    pltpu.sync_copy(rowsr_v, agg_sh.at[dstr_v], add=True)

    plsc.subcore_barrier()
    pltpu.sync_copy(agg_sh.at[pl.ds(sid * ROWS_PER_TILE, ROWS_PER_TILE)],
                    out_hbm.at[cid, pl.ds(sid * ROWS_PER_TILE, ROWS_PER_TILE)])


@functools.cache
def _make_agg():
    return pl.kernel(
        _agg_body,
        out_type=jax.ShapeDtypeStruct((NC, N_PAD, H), jnp.float32),
        mesh=plsc.VectorSubcoreMesh(core_axis_name="c", subcore_axis_name="s"),
        scratch_types=[
            pltpu.VMEM_SHARED((N_PAD, H), jnp.float32),
            pltpu.VMEM((CHUNK,), jnp.int32),
            pltpu.VMEM((CHUNK,), jnp.int32),
            pltpu.VMEM((CHUNK,), jnp.int32),
            pltpu.VMEM((CHUNK,), jnp.int32),
            pltpu.VMEM((CHUNK, H), jnp.float32),
            pltpu.VMEM((CHUNK, H), jnp.float32),
            pltpu.VMEM((REM,), jnp.int32),
            pltpu.VMEM((REM,), jnp.int32),
            pltpu.VMEM((REM, H), jnp.float32),
            pltpu.SemaphoreType.DMA,
            pltpu.SemaphoreType.DMA,
            pltpu.SemaphoreType.DMA,
            pltpu.SemaphoreType.DMA,
            pltpu.SemaphoreType.DMA,
        ],
    )


def _agg(x, src, dst, zeros):
    p = _make_agg()(x, src, dst, zeros)
    return p[:, :N]


RB = 1000
NB = N // RB


def _mlp_head_body(head, eps_ref, x_ref, p0_ref, p1_ref, Wa_ref, ba_ref,
                   Wb_ref, bb_ref, g_ref, beta_ref, Wl1_ref, bl1_ref,
                   Wl2_ref, bl2_ref, out_ref, h_all, acc_s, acc_q):
    i = pl.program_id(0)

    @pl.when(i < NB)
    def _compute():
        h = x_ref[...] * (1.0 + eps_ref[0]) + (p0_ref[...] + p1_ref[...])
        h = jnp.maximum(
            jnp.dot(h, Wa_ref[...], preferred_element_type=jnp.float32)
            + ba_ref[...], 0.0)
        h = jnp.maximum(
            jnp.dot(h, Wb_ref[...], preferred_element_type=jnp.float32)
            + bb_ref[...], 0.0)
        h_all[pl.ds(i * RB, RB), :] = h
        s = jnp.sum(h, axis=0, keepdims=True)
        q = jnp.sum(jnp.square(h), axis=0, keepdims=True)

        @pl.when(i == 0)
        def _():
            acc_s[...] = s
            acc_q[...] = q

        @pl.when(i > 0)
        def _():
            acc_s[...] = acc_s[...] + s
            acc_q[...] = acc_q[...] + q

    @pl.when(i >= NB)
    def _normalize():
        j = i - NB
        mean = acc_s[...] * (1.0 / N)
        var = acc_q[...] * (1.0 / N) - jnp.square(mean)
        h = h_all[pl.ds(j * RB, RB), :]
        h = (h - mean) * lax.rsqrt(var + BN_EPS) * g_ref[...] + beta_ref[...]
        if head:
            h = jnp.maximum(
                jnp.dot(h, Wl1_ref[...], preferred_element_type=jnp.float32)
                + bl1_ref[...], 0.0)
            out_ref[...] = (
                jnp.dot(h, Wl2_ref[...], preferred_element_type=jnp.float32)
                + bl2_ref[...])
        else:
            out_ref[...] = h


_SMEM1 = pl.BlockSpec(memory_space=pltpu.SMEM)


def _row_spec(width):
    return pl.BlockSpec((RB, width), lambda i: (jnp.minimum(i, NB - 1), 0))


def _full_spec(shape):
    nd = len(shape)
    return pl.BlockSpec(shape, (lambda i: (0, 0)) if nd == 2 else (lambda i: (0,)))


def _mlp_layers(head, eps, x, p0, p1, Wa, ba, Wb, bb, g, beta,
                Wl1, bl1, Wl2, bl2):
    out_w = C if head else H
    in_specs = [_SMEM1, _row_spec(H), _row_spec(H), _row_spec(H),
                _full_spec((H, H)), _full_spec((H,)), _full_spec((H, H)),
                _full_spec((H,)), _full_spec((H,)), _full_spec((H,)),
                _full_spec((H, H)), _full_spec((H,)), _full_spec((H, C)),
                _full_spec((C,))]
    return pl.pallas_call(
        functools.partial(_mlp_head_body, head),
        grid=(2 * NB,),
        out_shape=jax.ShapeDtypeStruct((N, out_w), jnp.float32),
        in_specs=in_specs,
        out_specs=pl.BlockSpec((RB, out_w), lambda i: (jnp.maximum(i - NB, 0), 0)),
        scratch_shapes=[pltpu.VMEM((N, H), jnp.float32),
                        pltpu.VMEM((1, H), jnp.float32),
                        pltpu.VMEM((1, H), jnp.float32)],
    )(eps.reshape(1), x, p0, p1, Wa, ba, Wb, bb, g, beta, Wl1, bl1, Wl2, bl2)


def _mlp_bn(eps, x, p0, p1, Wa, ba, Wb, bb, g, beta):
    dummy2 = jnp.zeros((H, H), jnp.float32)
    dummyc = jnp.zeros((H, C), jnp.float32)
    return _mlp_layers(False, eps, x, p0, p1, Wa, ba, Wb, bb, g, beta,
                       dummy2, jnp.zeros((H,), jnp.float32), dummyc,
                       jnp.zeros((C,), jnp.float32))


def _head(eps, x, p0, p1, Wa, ba, Wb, bb, g, beta, Wl1, bl1, Wl2, bl2):
    return _mlp_layers(True, eps, x, p0, p1, Wa, ba, Wb, bb, g, beta,
                       Wl1, bl1, Wl2, bl2)


def kernel(x, edge_index, eps0, W0a, b0a, W0b, b0b, g0, beta0,
           eps1, W1a, b1a, W1b, b1b, g1, beta1,
           eps2, W2a, b2a, W2b, b2b, g2, beta2, Wl1, bl1, Wl2, bl2):
    src = edge_index[0]
    dst = edge_index[1]
    zeros = jnp.zeros((N_PAD, H), jnp.float32)

    p = _agg(x, src, dst, zeros)
    h = _mlp_bn(eps0, x, p[0], p[1], W0a, b0a, W0b, b0b, g0, beta0)
    p = _agg(h, src, dst, zeros)
    h = _mlp_bn(eps1, h, p[0], p[1], W1a, b1a, W1b, b1b, g1, beta1)
    p = _agg(h, src, dst, zeros)
    return _head(eps2, h, p[0], p[1], W2a, b2a, W2b, b2b, g2, beta2,
                 Wl1, bl1, Wl2, bl2)


# speedup vs baseline: 1.0311x; 1.0311x over previous
import functools

import jax
import jax.numpy as jnp
from jax import lax
from jax.experimental import pallas as pl
from jax.experimental.pallas import tpu as pltpu
from jax.experimental.pallas import tpu_sc as plsc

N = 10000
E = 320000
H = 128
C = 40
BN_EPS = 1e-5

NC = 2
NS = 16
NW = NC * NS
EPW = E // NW
CHUNK = 128
FULL_CHUNKS = EPW // CHUNK
REM = EPW - FULL_CHUNKS * CHUNK
N_PAD = 10112
ROWS_PER_TILE = N_PAD // NS


def _agg_body(x_hbm, src_hbm, dst_hbm, zeros_hbm, out_hbm,
              agg_sh, src_v0, dst_v0, src_v1, dst_v1,
              rows_v0, rows_v1, srcr_v, dstr_v, rowsr_v,
              sem_g0, sem_g1, sem_i0, sem_i1, sem_r):
    cid = lax.axis_index("c")
    sid = lax.axis_index("s")
    wid = sid * NC + cid
    base = wid * EPW

    src_b = (src_v0, src_v1)
    dst_b = (dst_v0, dst_v1)
    rows_b = (rows_v0, rows_v1)
    sem_g = (sem_g0, sem_g1)
    sem_i = (sem_i0, sem_i1)

    pltpu.sync_copy(zeros_hbm.at[pl.ds(sid * ROWS_PER_TILE, ROWS_PER_TILE)],
                    agg_sh.at[pl.ds(sid * ROWS_PER_TILE, ROWS_PER_TILE)])

    def issue_idx(c, b):
        off = base + c * CHUNK
        pltpu.async_copy(src_hbm.at[pl.ds(off, CHUNK)], src_b[b], sem_i[b])
        pltpu.async_copy(dst_hbm.at[pl.ds(off, CHUNK)], dst_b[b], sem_i[b])

    def wait_idx(c, b):
        off = base + c * CHUNK
        pltpu.make_async_copy(src_hbm.at[pl.ds(off, CHUNK)], src_b[b], sem_i[b]).wait()
        pltpu.make_async_copy(dst_hbm.at[pl.ds(off, CHUNK)], dst_b[b], sem_i[b]).wait()

    def start_gather(b):
        pltpu.async_copy(x_hbm.at[src_b[b]], rows_b[b], sem_g[b])

    def wait_gather(b):
        pltpu.make_async_copy(x_hbm.at[src_b[b]], rows_b[b], sem_g[b]).wait()

    def scatter(b):
        pltpu.sync_copy(rows_b[b], agg_sh.at[dst_b[b]], add=True)

    issue_idx(0, 0)
    issue_idx(1, 1)
    plsc.subcore_barrier()
    wait_idx(0, 0)
    start_gather(0)

    def half_step(c, b):
        wait_idx(c + 1, 1 - b)
        start_gather(1 - b)
        wait_gather(b)
        scatter(b)
        issue_idx(c + 2, b)

    def pair_step(i, carry):
        half_step(2 * i, 0)
        half_step(2 * i + 1, 1)
        return carry

    lax.fori_loop(0, FULL_CHUNKS // 2 - 1, pair_step, 0)

    wait_idx(FULL_CHUNKS - 1, 1)
    start_gather(1)
    wait_gather(0)
    scatter(0)
    wait_gather(1)
    scatter(1)

    off = base + FULL_CHUNKS * CHUNK
    pltpu.sync_copy(src_hbm.at[pl.ds(off, REM)], srcr_v)
    pltpu.sync_copy(dst_hbm.at[pl.ds(off, REM)], dstr_v)
    pltpu.async_copy(x_hbm.at[srcr_v], rowsr_v, sem_r).wait()
    pltpu.sync_copy(rowsr_v, agg_sh.at[dstr_v], add=True)

    plsc.subcore_barrier()
    pltpu.sync_copy(agg_sh.at[pl.ds(sid * ROWS_PER_TILE, ROWS_PER_TILE)],
                    out_hbm.at[cid, pl.ds(sid * ROWS_PER_TILE, ROWS_PER_TILE)])


@functools.cache
def _make_agg():
    return pl.kernel(
        _agg_body,
        out_type=jax.ShapeDtypeStruct((NC, N_PAD, H), jnp.float32),
        mesh=plsc.VectorSubcoreMesh(core_axis_name="c", subcore_axis_name="s"),
        scratch_types=[
            pltpu.VMEM_SHARED((N_PAD, H), jnp.float32),
            pltpu.VMEM((CHUNK,), jnp.int32),
            pltpu.VMEM((CHUNK,), jnp.int32),
            pltpu.VMEM((CHUNK,), jnp.int32),
            pltpu.VMEM((CHUNK,), jnp.int32),
            pltpu.VMEM((CHUNK, H), jnp.float32),
            pltpu.VMEM((CHUNK, H), jnp.float32),
            pltpu.VMEM((REM,), jnp.int32),
            pltpu.VMEM((REM,), jnp.int32),
            pltpu.VMEM((REM, H), jnp.float32),
            pltpu.SemaphoreType.DMA,
            pltpu.SemaphoreType.DMA,
            pltpu.SemaphoreType.DMA,
            pltpu.SemaphoreType.DMA,
            pltpu.SemaphoreType.DMA,
        ],
    )


def _agg(x, src, dst, zeros):
    p = _make_agg()(x, src, dst, zeros)
    return p[:, :N]


def _mlp_bn_body(eps_ref, x_ref, p0_ref, p1_ref, Wa_ref, ba_ref, Wb_ref,
                 bb_ref, g_ref, beta_ref, out_ref):
    h = x_ref[...] * (1.0 + eps_ref[0]) + (p0_ref[...] + p1_ref[...])
    h = jnp.maximum(jnp.dot(h, Wa_ref[...], preferred_element_type=jnp.float32)
                    + ba_ref[...], 0.0)
    h = jnp.maximum(jnp.dot(h, Wb_ref[...], preferred_element_type=jnp.float32)
                    + bb_ref[...], 0.0)
    mean = jnp.mean(h, axis=0, keepdims=True)
    var = jnp.mean(jnp.square(h - mean), axis=0, keepdims=True)
    out_ref[...] = (h - mean) * lax.rsqrt(var + BN_EPS) * g_ref[...] + beta_ref[...]


def _head_body(eps_ref, x_ref, p0_ref, p1_ref, Wa_ref, ba_ref, Wb_ref,
               bb_ref, g_ref, beta_ref, Wl1_ref, bl1_ref, Wl2_ref, bl2_ref,
               out_ref):
    h = x_ref[...] * (1.0 + eps_ref[0]) + (p0_ref[...] + p1_ref[...])
    h = jnp.maximum(jnp.dot(h, Wa_ref[...], preferred_element_type=jnp.float32)
                    + ba_ref[...], 0.0)
    h = jnp.maximum(jnp.dot(h, Wb_ref[...], preferred_element_type=jnp.float32)
                    + bb_ref[...], 0.0)
    mean = jnp.mean(h, axis=0, keepdims=True)
    var = jnp.mean(jnp.square(h - mean), axis=0, keepdims=True)
    h = (h - mean) * lax.rsqrt(var + BN_EPS) * g_ref[...] + beta_ref[...]
    h = jnp.maximum(jnp.dot(h, Wl1_ref[...], preferred_element_type=jnp.float32)
                    + bl1_ref[...], 0.0)
    out_ref[...] = (jnp.dot(h, Wl2_ref[...], preferred_element_type=jnp.float32)
                    + bl2_ref[...])


_SMEM1 = pl.BlockSpec(memory_space=pltpu.SMEM)


def _mlp_bn(eps, x, p0, p1, Wa, ba, Wb, bb, g, beta):
    return pl.pallas_call(
        _mlp_bn_body,
        out_shape=jax.ShapeDtypeStruct((N, H), jnp.float32),
        in_specs=[_SMEM1] + [pl.BlockSpec()] * 9,
        out_specs=pl.BlockSpec(),
    )(eps.reshape(1), x, p0, p1, Wa, ba, Wb, bb, g, beta)


def _head(eps, x, p0, p1, Wa, ba, Wb, bb, g, beta, Wl1, bl1, Wl2, bl2):
    return pl.pallas_call(
        _head_body,
        out_shape=jax.ShapeDtypeStruct((N, C), jnp.float32),
        in_specs=[_SMEM1] + [pl.BlockSpec()] * 13,
        out_specs=pl.BlockSpec(),
    )(eps.reshape(1), x, p0, p1, Wa, ba, Wb, bb, g, beta, Wl1, bl1, Wl2, bl2)


def kernel(x, edge_index, eps0, W0a, b0a, W0b, b0b, g0, beta0,
           eps1, W1a, b1a, W1b, b1b, g1, beta1,
           eps2, W2a, b2a, W2b, b2b, g2, beta2, Wl1, bl1, Wl2, bl2):
    src = edge_index[0]
    dst = edge_index[1]
    zeros = jnp.zeros((N_PAD, H), jnp.float32)

    p = _agg(x, src, dst, zeros)
    h = _mlp_bn(eps0, x, p[0], p[1], W0a, b0a, W0b, b0b, g0, beta0)
    p = _agg(h, src, dst, zeros)
    h = _mlp_bn(eps1, h, p[0], p[1], W1a, b1a, W1b, b1b, g1, beta1)
    p = _agg(h, src, dst, zeros)
    return _head(eps2, h, p[0], p[1], W2a, b2a, W2b, b2b, g2, beta2,
                 Wl1, bl1, Wl2, bl2)


# speedup vs baseline: 1.0762x; 1.0437x over previous
import functools

import jax
import jax.numpy as jnp
from jax import lax
from jax.experimental import pallas as pl
from jax.experimental.pallas import tpu as pltpu
from jax.experimental.pallas import tpu_sc as plsc

N = 10000
E = 320000
H = 128
C = 40
BN_EPS = 1e-5

NC = 2
NS = 16
NW = NC * NS
EPW = E // NW
CHUNK = 128
FULL_CHUNKS = EPW // CHUNK
REM = EPW - FULL_CHUNKS * CHUNK
N_PAD = 10112
ROWS_PER_TILE = N_PAD // NS


def _agg_body(x_hbm, src_hbm, dst_hbm, zeros_hbm, out_hbm,
              agg_sh, src_v0, dst_v0, src_v1, dst_v1,
              rows_v0, rows_v1, srcr_v, dstr_v, rowsr_v,
              sem_g0, sem_g1, sem_i0, sem_i1, sem_r):
    cid = lax.axis_index("c")
    sid = lax.axis_index("s")
    wid = sid * NC + cid
    base = wid * EPW

    src_b = (src_v0, src_v1)
    dst_b = (dst_v0, dst_v1)
    rows_b = (rows_v0, rows_v1)
    sem_g = (sem_g0, sem_g1)
    sem_i = (sem_i0, sem_i1)

    pltpu.sync_copy(zeros_hbm.at[pl.ds(sid * ROWS_PER_TILE, ROWS_PER_TILE)],
                    agg_sh.at[pl.ds(sid * ROWS_PER_TILE, ROWS_PER_TILE)])

    def issue_idx(c, b):
        off = base + c * CHUNK
        pltpu.async_copy(src_hbm.at[pl.ds(off, CHUNK)], src_b[b], sem_i[b])
        pltpu.async_copy(dst_hbm.at[pl.ds(off, CHUNK)], dst_b[b], sem_i[b])

    def wait_idx(c, b):
        off = base + c * CHUNK
        pltpu.make_async_copy(src_hbm.at[pl.ds(off, CHUNK)], src_b[b], sem_i[b]).wait()
        pltpu.make_async_copy(dst_hbm.at[pl.ds(off, CHUNK)], dst_b[b], sem_i[b]).wait()

    def start_gather(b):
        pltpu.async_copy(x_hbm.at[src_b[b]], rows_b[b], sem_g[b])

    def wait_gather(b):
        pltpu.make_async_copy(x_hbm.at[src_b[b]], rows_b[b], sem_g[b]).wait()

    def scatter(b):
        pltpu.sync_copy(rows_b[b], agg_sh.at[dst_b[b]], add=True)

    issue_idx(0, 0)
    issue_idx(1, 1)
    plsc.subcore_barrier()
    wait_idx(0, 0)
    start_gather(0)

    def half_step(c, b):
        wait_idx(c + 1, 1 - b)
        start_gather(1 - b)
        wait_gather(b)
        scatter(b)
        issue_idx(c + 2, b)

    def pair_step(i, carry):
        half_step(2 * i, 0)
        half_step(2 * i + 1, 1)
        return carry

    lax.fori_loop(0, FULL_CHUNKS // 2 - 1, pair_step, 0)

    wait_idx(FULL_CHUNKS - 1, 1)
    start_gather(1)
    wait_gather(0)
    scatter(0)
    wait_gather(1)
    scatter(1)

    off = base + FULL_CHUNKS * CHUNK
    pltpu.sync_copy(src_hbm.at[pl.ds(off, REM)], srcr_v)
    pltpu.sync_copy(dst_hbm.at[pl.ds(off, REM)], dstr_v)
    pltpu.async_copy(x_hbm.at[srcr_v], rowsr_v, sem_r).wait()
    pltpu.sync_copy(rowsr_v, agg_sh.at[dstr_v], add=True)

    plsc.subcore_barrier()
    pltpu.sync_copy(agg_sh.at[pl.ds(sid * ROWS_PER_TILE, ROWS_PER_TILE)],
                    out_hbm.at[cid, pl.ds(sid * ROWS_PER_TILE, ROWS_PER_TILE)])


@functools.cache
def _make_agg():
    return pl.kernel(
        _agg_body,
        out_type=jax.ShapeDtypeStruct((NC, N_PAD, H), jnp.float32),
        mesh=plsc.VectorSubcoreMesh(core_axis_name="c", subcore_axis_name="s"),
        scratch_types=[
            pltpu.VMEM_SHARED((N_PAD, H), jnp.float32),
            pltpu.VMEM((CHUNK,), jnp.int32),
            pltpu.VMEM((CHUNK,), jnp.int32),
            pltpu.VMEM((CHUNK,), jnp.int32),
            pltpu.VMEM((CHUNK,), jnp.int32),
            pltpu.VMEM((CHUNK, H), jnp.float32),
            pltpu.VMEM((CHUNK, H), jnp.float32),
            pltpu.VMEM((REM,), jnp.int32),
            pltpu.VMEM((REM,), jnp.int32),
            pltpu.VMEM((REM, H), jnp.float32),
            pltpu.SemaphoreType.DMA,
            pltpu.SemaphoreType.DMA,
            pltpu.SemaphoreType.DMA,
            pltpu.SemaphoreType.DMA,
            pltpu.SemaphoreType.DMA,
        ],
    )


def _agg(x, src, dst, zeros):
    return _make_agg()(x, src, dst, zeros)


def _mlp_bn_body(eps_ref, x_ref, p_ref, Wa_ref, ba_ref, Wb_ref,
                 bb_ref, g_ref, beta_ref, out_ref):
    h = x_ref[...] * (1.0 + eps_ref[0]) + (p_ref[0] + p_ref[1])
    h = jnp.maximum(jnp.dot(h, Wa_ref[...], preferred_element_type=jnp.float32)
                    + ba_ref[...], 0.0)
    h = jnp.maximum(jnp.dot(h, Wb_ref[...], preferred_element_type=jnp.float32)
                    + bb_ref[...], 0.0)
    mean = jnp.mean(h, axis=0, keepdims=True)
    var = jnp.mean(jnp.square(h - mean), axis=0, keepdims=True)
    out_ref[...] = (h - mean) * lax.rsqrt(var + BN_EPS) * g_ref[...] + beta_ref[...]


def _head_body(eps_ref, x_ref, p_ref, Wa_ref, ba_ref, Wb_ref,
               bb_ref, g_ref, beta_ref, Wl1_ref, bl1_ref, Wl2_ref, bl2_ref,
               out_ref):
    h = x_ref[...] * (1.0 + eps_ref[0]) + (p_ref[0] + p_ref[1])
    h = jnp.maximum(jnp.dot(h, Wa_ref[...], preferred_element_type=jnp.float32)
                    + ba_ref[...], 0.0)
    h = jnp.maximum(jnp.dot(h, Wb_ref[...], preferred_element_type=jnp.float32)
                    + bb_ref[...], 0.0)
    mean = jnp.mean(h, axis=0, keepdims=True)
    var = jnp.mean(jnp.square(h - mean), axis=0, keepdims=True)
    h = (h - mean) * lax.rsqrt(var + BN_EPS) * g_ref[...] + beta_ref[...]
    h = jnp.maximum(jnp.dot(h, Wl1_ref[...], preferred_element_type=jnp.float32)
                    + bl1_ref[...], 0.0)
    out_ref[...] = (jnp.dot(h, Wl2_ref[...], preferred_element_type=jnp.float32)
                    + bl2_ref[...])


_SMEM1 = pl.BlockSpec(memory_space=pltpu.SMEM)
_PSPEC = pl.BlockSpec((NC, N, H), lambda i: (0, 0, 0))


def _mlp_bn(eps, x, p, Wa, ba, Wb, bb, g, beta):
    return pl.pallas_call(
        _mlp_bn_body,
        grid=(1,),
        out_shape=jax.ShapeDtypeStruct((N, H), jnp.float32),
        in_specs=[_SMEM1, pl.BlockSpec(), _PSPEC] + [pl.BlockSpec()] * 6,
        out_specs=pl.BlockSpec(),
    )(eps.reshape(1), x, p, Wa, ba, Wb, bb, g, beta)


def _head(eps, x, p, Wa, ba, Wb, bb, g, beta, Wl1, bl1, Wl2, bl2):
    return pl.pallas_call(
        _head_body,
        grid=(1,),
        out_shape=jax.ShapeDtypeStruct((N, C), jnp.float32),
        in_specs=[_SMEM1, pl.BlockSpec(), _PSPEC] + [pl.BlockSpec()] * 10,
        out_specs=pl.BlockSpec(),
    )(eps.reshape(1), x, p, Wa, ba, Wb, bb, g, beta, Wl1, bl1, Wl2, bl2)


def kernel(x, edge_index, eps0, W0a, b0a, W0b, b0b, g0, beta0,
           eps1, W1a, b1a, W1b, b1b, g1, beta1,
           eps2, W2a, b2a, W2b, b2b, g2, beta2, Wl1, bl1, Wl2, bl2):
    src = edge_index[0]
    dst = edge_index[1]
    zeros = jnp.zeros((N_PAD, H), jnp.float32)

    p = _agg(x, src, dst, zeros)
    h = _mlp_bn(eps0, x, p, W0a, b0a, W0b, b0b, g0, beta0)
    p = _agg(h, src, dst, zeros)
    h = _mlp_bn(eps1, h, p, W1a, b1a, W1b, b1b, g1, beta1)
    p = _agg(h, src, dst, zeros)
    return _head(eps2, h, p, W2a, b2a, W2b, b2b, g2, beta2,
                 Wl1, bl1, Wl2, bl2)


# speedup vs baseline: 1.0954x; 1.0179x over previous
import functools

import jax
import jax.numpy as jnp
from jax import lax
from jax.experimental import pallas as pl
from jax.experimental.pallas import tpu as pltpu
from jax.experimental.pallas import tpu_sc as plsc

N = 10000
E = 320000
H = 128
C = 40
BN_EPS = 1e-5

NC = 2
NS = 16
NW = NC * NS
EPW = E // NW
CHUNK = 128
FULL_CHUNKS = EPW // CHUNK
REM = EPW - FULL_CHUNKS * CHUNK
N_PAD = 10112
ROWS_PER_TILE = N_PAD // NS


def _agg_body(x_hbm, src_hbm, dst_hbm, zeros_hbm, out_hbm,
              agg_sh, src_v0, dst_v0, src_v1, dst_v1,
              rows_v0, rows_v1, srcr_v, dstr_v, rowsr_v,
              sem_g0, sem_g1, sem_i0, sem_i1, sem_r):
    cid = lax.axis_index("c")
    sid = lax.axis_index("s")
    wid = sid * NC + cid
    base = wid * EPW

    src_b = (src_v0, src_v1)
    dst_b = (dst_v0, dst_v1)
    rows_b = (rows_v0, rows_v1)
    sem_g = (sem_g0, sem_g1)
    sem_i = (sem_i0, sem_i1)

    def issue_idx(c, b):
        off = base + c * CHUNK
        pltpu.async_copy(src_hbm.at[pl.ds(off, CHUNK)], src_b[b], sem_i[b])
        pltpu.async_copy(dst_hbm.at[pl.ds(off, CHUNK)], dst_b[b], sem_i[b])

    def wait_idx(c, b):
        off = base + c * CHUNK
        pltpu.make_async_copy(src_hbm.at[pl.ds(off, CHUNK)], src_b[b], sem_i[b]).wait()
        pltpu.make_async_copy(dst_hbm.at[pl.ds(off, CHUNK)], dst_b[b], sem_i[b]).wait()

    def start_gather(b):
        pltpu.async_copy(x_hbm.at[src_b[b]], rows_b[b], sem_g[b])

    def wait_gather(b):
        pltpu.make_async_copy(x_hbm.at[src_b[b]], rows_b[b], sem_g[b]).wait()

    def scatter(b):
        pltpu.sync_copy(rows_b[b], agg_sh.at[dst_b[b]], add=True)

    issue_idx(0, 0)
    issue_idx(1, 1)
    wait_idx(0, 0)
    start_gather(0)
    wait_idx(1, 1)
    start_gather(1)
    pltpu.sync_copy(zeros_hbm.at[pl.ds(sid * ROWS_PER_TILE, ROWS_PER_TILE)],
                    agg_sh.at[pl.ds(sid * ROWS_PER_TILE, ROWS_PER_TILE)])
    plsc.subcore_barrier()

    def half_step(c, b, warm=False):
        if not warm:
            wait_idx(c + 1, 1 - b)
            start_gather(1 - b)
        wait_gather(b)
        scatter(b)
        issue_idx(c + 2, b)

    half_step(0, 0, warm=True)
    half_step(1, 1)

    def pair_step(i, carry):
        half_step(2 * i + 2, 0)
        half_step(2 * i + 3, 1)
        return carry

    lax.fori_loop(0, FULL_CHUNKS // 2 - 2, pair_step, 0)

    wait_idx(FULL_CHUNKS - 1, 1)
    start_gather(1)
    wait_gather(0)
    scatter(0)
    wait_gather(1)
    scatter(1)

    off = base + FULL_CHUNKS * CHUNK
    pltpu.sync_copy(src_hbm.at[pl.ds(off, REM)], srcr_v)
    pltpu.sync_copy(dst_hbm.at[pl.ds(off, REM)], dstr_v)
    pltpu.async_copy(x_hbm.at[srcr_v], rowsr_v, sem_r).wait()
    pltpu.sync_copy(rowsr_v, agg_sh.at[dstr_v], add=True)

    plsc.subcore_barrier()
    pltpu.sync_copy(agg_sh.at[pl.ds(sid * ROWS_PER_TILE, ROWS_PER_TILE)],
                    out_hbm.at[cid, pl.ds(sid * ROWS_PER_TILE, ROWS_PER_TILE)])


@functools.cache
def _make_agg():
    return pl.kernel(
        _agg_body,
        out_type=jax.ShapeDtypeStruct((NC, N_PAD, H), jnp.float32),
        mesh=plsc.VectorSubcoreMesh(core_axis_name="c", subcore_axis_name="s"),
        scratch_types=[
            pltpu.VMEM_SHARED((N_PAD, H), jnp.float32),
            pltpu.VMEM((CHUNK,), jnp.int32),
            pltpu.VMEM((CHUNK,), jnp.int32),
            pltpu.VMEM((CHUNK,), jnp.int32),
            pltpu.VMEM((CHUNK,), jnp.int32),
            pltpu.VMEM((CHUNK, H), jnp.float32),
            pltpu.VMEM((CHUNK, H), jnp.float32),
            pltpu.VMEM((REM,), jnp.int32),
            pltpu.VMEM((REM,), jnp.int32),
            pltpu.VMEM((REM, H), jnp.float32),
            pltpu.SemaphoreType.DMA,
            pltpu.SemaphoreType.DMA,
            pltpu.SemaphoreType.DMA,
            pltpu.SemaphoreType.DMA,
            pltpu.SemaphoreType.DMA,
        ],
    )


def _agg(x, src, dst, zeros):
    return _make_agg()(x, src, dst, zeros)


def _mlp_bn_body(eps_ref, x_ref, p_ref, Wa_ref, ba_ref, Wb_ref,
                 bb_ref, g_ref, beta_ref, out_ref):
    h = x_ref[...] * (1.0 + eps_ref[0]) + (p_ref[0] + p_ref[1])
    h = jnp.maximum(jnp.dot(h, Wa_ref[...], preferred_element_type=jnp.float32)
                    + ba_ref[...], 0.0)
    h = jnp.maximum(jnp.dot(h, Wb_ref[...], preferred_element_type=jnp.float32)
                    + bb_ref[...], 0.0)
    mean = jnp.mean(h, axis=0, keepdims=True)
    var = jnp.mean(jnp.square(h - mean), axis=0, keepdims=True)
    out_ref[...] = (h - mean) * lax.rsqrt(var + BN_EPS) * g_ref[...] + beta_ref[...]


def _head_body(eps_ref, x_ref, p_ref, Wa_ref, ba_ref, Wb_ref,
               bb_ref, g_ref, beta_ref, Wl1_ref, bl1_ref, Wl2_ref, bl2_ref,
               out_ref):
    h = x_ref[...] * (1.0 + eps_ref[0]) + (p_ref[0] + p_ref[1])
    h = jnp.maximum(jnp.dot(h, Wa_ref[...], preferred_element_type=jnp.float32)
                    + ba_ref[...], 0.0)
    h = jnp.maximum(jnp.dot(h, Wb_ref[...], preferred_element_type=jnp.float32)
                    + bb_ref[...], 0.0)
    mean = jnp.mean(h, axis=0, keepdims=True)
    var = jnp.mean(jnp.square(h - mean), axis=0, keepdims=True)
    h = (h - mean) * lax.rsqrt(var + BN_EPS) * g_ref[...] + beta_ref[...]
    h = jnp.maximum(jnp.dot(h, Wl1_ref[...], preferred_element_type=jnp.float32)
                    + bl1_ref[...], 0.0)
    out_ref[...] = (jnp.dot(h, Wl2_ref[...], preferred_element_type=jnp.float32)
                    + bl2_ref[...])


_SMEM1 = pl.BlockSpec(memory_space=pltpu.SMEM)
_PSPEC = pl.BlockSpec((NC, N, H), lambda i: (0, 0, 0))


def _mlp_bn(eps, x, p, Wa, ba, Wb, bb, g, beta):
    return pl.pallas_call(
        _mlp_bn_body,
        grid=(1,),
        out_shape=jax.ShapeDtypeStruct((N, H), jnp.float32),
        in_specs=[_SMEM1, pl.BlockSpec(), _PSPEC] + [pl.BlockSpec()] * 6,
        out_specs=pl.BlockSpec(),
    )(eps.reshape(1), x, p, Wa, ba, Wb, bb, g, beta)


def _head(eps, x, p, Wa, ba, Wb, bb, g, beta, Wl1, bl1, Wl2, bl2):
    return pl.pallas_call(
        _head_body,
        grid=(1,),
        out_shape=jax.ShapeDtypeStruct((N, C), jnp.float32),
        in_specs=[_SMEM1, pl.BlockSpec(), _PSPEC] + [pl.BlockSpec()] * 10,
        out_specs=pl.BlockSpec(),
    )(eps.reshape(1), x, p, Wa, ba, Wb, bb, g, beta, Wl1, bl1, Wl2, bl2)


def kernel(x, edge_index, eps0, W0a, b0a, W0b, b0b, g0, beta0,
           eps1, W1a, b1a, W1b, b1b, g1, beta1,
           eps2, W2a, b2a, W2b, b2b, g2, beta2, Wl1, bl1, Wl2, bl2):
    src = edge_index[0]
    dst = edge_index[1]
    zeros = jnp.zeros((N_PAD, H), jnp.float32)

    p = _agg(x, src, dst, zeros)
    h = _mlp_bn(eps0, x, p, W0a, b0a, W0b, b0b, g0, beta0)
    p = _agg(h, src, dst, zeros)
    h = _mlp_bn(eps1, h, p, W1a, b1a, W1b, b1b, g1, beta1)
    p = _agg(h, src, dst, zeros)
    return _head(eps2, h, p, W2a, b2a, W2b, b2b, g2, beta2,
                 Wl1, bl1, Wl2, bl2)
